# baseline (speedup 1.0000x reference)
import functools

import jax
import jax.numpy as jnp
from jax import lax
from jax.experimental import pallas as pl
from jax.experimental.pallas import tpu as pltpu
from jax.experimental.pallas import tpu_sc as plsc

NUM_EMB = 1_000_000
DIM = 32
BATCH = 16384
HIST = 50
B_TOTAL = BATCH * HIST

_INFO = plsc.get_sparse_core_info()
_NC, _NS = _INFO.num_cores, _INFO.num_subcores
NW = _NC * _NS
BPW = BATCH // NW
NBUF = 5
NOUTER = HIST // NBUF

_mesh = plsc.VectorSubcoreMesh(core_axis_name="c", subcore_axis_name="s")


@functools.partial(
    pl.kernel,
    out_type=jax.ShapeDtypeStruct((B_TOTAL, DIM), jnp.float32),
    mesh=_mesh,
    scratch_types=[
        pltpu.VMEM((HIST, BPW), jnp.int32),
        [pltpu.VMEM((BPW, DIM), jnp.float32) for _ in range(NBUF)],
        [pltpu.SemaphoreType.DMA for _ in range(NBUF)],
    ],
    compiler_params=pltpu.CompilerParams(use_tc_tiling_on_sc=False),
)
def _gather_rows(table_hbm, idx_hbm, out_hbm, idx_v, bufs, sems):
    wid = lax.axis_index("s") * _NC + lax.axis_index("c")
    b0 = wid * BPW
    pltpu.sync_copy(idx_hbm.at[:, pl.ds(b0, BPW)], idx_v)

    @pl.loop(0, NOUTER)
    def _outer(j):
        h0 = j * NBUF
        gathers = []
        for p in range(NBUF):
            gathers.append(
                pltpu.async_copy(
                    table_hbm.at[idx_v.at[h0 + p]], bufs[p], sems[p]
                )
            )
        for p in range(NBUF):
            gathers[p].wait()
            pltpu.sync_copy(
                bufs[p], out_hbm.at[pl.ds((h0 + p) * BATCH + b0, BPW)]
            )


@functools.partial(
    pl.kernel,
    out_type=jax.ShapeDtypeStruct((HIST, DIM, BATCH), jnp.float32),
    mesh=_mesh,
    scratch_types=[
        pltpu.VMEM((BPW * DIM,), jnp.float32),
        pltpu.VMEM((8, BPW), jnp.float32),
        pltpu.SemaphoreType.DMA,
    ],
    compiler_params=pltpu.CompilerParams(
        use_tc_tiling_on_sc=True, needs_layout_passes=False
    ),
)
def _format_out(in_hbm, out_hbm, inb, og, sem):
    wid = lax.axis_index("s") * _NC + lax.axis_index("c")
    b0 = wid * BPW
    lane = lax.iota(jnp.int32, 16) * DIM

    @pl.loop(0, HIST)
    def _h(h):
        pltpu.sync_copy(
            in_hbm.at[pl.ds((h * BATCH + b0) * DIM, BPW * DIM)], inb
        )
        for g in range(DIM // 8):
            for ds in range(8):
                d = 8 * g + ds
                for lb in range(BPW // 16):
                    idx = lane + (lb * 16 * DIM + d)
                    og[ds, pl.ds(lb * 16, 16)] = plsc.load_gather(inb, [idx])
            pltpu.sync_copy(og, out_hbm.at[h, pl.ds(8 * g, 8), pl.ds(b0, BPW)])


def kernel(x, table):
    idx_t = jnp.minimum(x.T, NUM_EMB - 1)
    rows = _gather_rows(table, idx_t)
    out_f = _format_out(rows.reshape(-1))
    return out_f.transpose(2, 0, 1)


# speedup vs baseline: 1.1829x; 1.1829x over previous
import functools

import jax
import jax.numpy as jnp
from jax import lax
from jax.experimental import pallas as pl
from jax.experimental.pallas import tpu as pltpu
from jax.experimental.pallas import tpu_sc as plsc

NUM_EMB = 1_000_000
DIM = 32
BATCH = 16384
HIST = 50
B_TOTAL = BATCH * HIST

_INFO = plsc.get_sparse_core_info()
_NC, _NS = _INFO.num_cores, _INFO.num_subcores
NW = _NC * _NS
BPW = BATCH // NW
NBUF = 5
NOUTER = HIST // NBUF

_mesh = plsc.VectorSubcoreMesh(core_axis_name="c", subcore_axis_name="s")


@functools.partial(
    pl.kernel,
    out_type=jax.ShapeDtypeStruct((B_TOTAL, DIM), jnp.float32),
    mesh=_mesh,
    scratch_types=[
        pltpu.VMEM((HIST, BPW), jnp.int32),
        [pltpu.VMEM((BPW, DIM), jnp.float32) for _ in range(NBUF)],
        [pltpu.SemaphoreType.DMA for _ in range(NBUF)],
    ],
    compiler_params=pltpu.CompilerParams(use_tc_tiling_on_sc=False),
)
def _gather_rows(table_hbm, idx_hbm, out_hbm, idx_v, bufs, sems):
    wid = lax.axis_index("s") * _NC + lax.axis_index("c")
    b0 = wid * BPW
    pltpu.sync_copy(idx_hbm.at[:, pl.ds(b0, BPW)], idx_v)

    @pl.loop(0, NOUTER)
    def _outer(j):
        h0 = j * NBUF
        gathers = []
        for p in range(NBUF):
            gathers.append(
                pltpu.async_copy(
                    table_hbm.at[idx_v.at[h0 + p]], bufs[p], sems[p]
                )
            )
        for p in range(NBUF):
            gathers[p].wait()
            pltpu.sync_copy(
                bufs[p], out_hbm.at[pl.ds((h0 + p) * BATCH + b0, BPW)]
            )


@functools.partial(
    pl.kernel,
    out_type=jax.ShapeDtypeStruct((HIST, DIM, BATCH), jnp.float32),
    mesh=_mesh,
    scratch_types=[
        [pltpu.VMEM((BPW * DIM,), jnp.float32) for _ in range(2)],
        [pltpu.VMEM((DIM, BPW), jnp.float32) for _ in range(2)],
        [pltpu.SemaphoreType.DMA for _ in range(2)],
        [pltpu.SemaphoreType.DMA for _ in range(2)],
    ],
    compiler_params=pltpu.CompilerParams(
        use_tc_tiling_on_sc=True, needs_layout_passes=False
    ),
)
def _format_out(in_hbm, out_hbm, inb, og, sem_in, sem_og):
    wid = lax.axis_index("s") * _NC + lax.axis_index("c")
    b0 = wid * BPW
    lane32 = lax.iota(jnp.int32, 16) * DIM

    def in_slice(h):
        return in_hbm.at[pl.ds((h * BATCH + b0) * DIM, BPW * DIM)]

    def out_slice(h):
        return out_hbm.at[h, :, pl.ds(b0, BPW)]

    for ph in range(2):
        pltpu.async_copy(in_slice(ph), inb[ph], sem_in[ph])

    @pl.loop(0, HIST, step=2)
    def _hh(hh):
        for ph in range(2):
            h = hh + ph
            pltpu.make_async_copy(in_slice(h), inb[ph], sem_in[ph]).wait()

            @pl.when(h >= 2)
            def _():
                pltpu.make_async_copy(og[ph], out_slice(0), sem_og[ph]).wait()

            for d in range(DIM):
                vs = [
                    plsc.load_gather(inb[ph], [lane32 + (lb * 16 * DIM + d)])
                    for lb in range(BPW // 16)
                ]
                for lb in range(BPW // 16):
                    og[ph][d, pl.ds(lb * 16, 16)] = vs[lb]

            pltpu.async_copy(og[ph], out_slice(h), sem_og[ph])

            @pl.when(h + 2 < HIST)
            def _():
                pltpu.async_copy(in_slice(h + 2), inb[ph], sem_in[ph])

    for ph in range(2):
        pltpu.make_async_copy(og[ph], out_slice(0), sem_og[ph]).wait()


def kernel(x, table):
    idx_t = jnp.minimum(x.T, NUM_EMB - 1)
    rows = _gather_rows(table, idx_t)
    out_f = _format_out(rows.reshape(-1))
    return out_f.transpose(2, 0, 1)
